# Initial kernel scaffold; baseline (speedup 1.0000x reference)
import jax
import jax.numpy as jnp
from jax.experimental import pallas as pl


def kernel(points, W_rel, b_rel, W_dist, b_dist, emb, W_den, b_den):
    raise NotImplementedError("write your pallas kernel here")


# speedup vs baseline: 7.9151x; 7.9151x over previous
import functools

import jax
import jax.numpy as jnp
from jax import lax
from jax.experimental import pallas as pl
from jax.experimental.pallas import tpu as pltpu
from jax.experimental.pallas import tpu_sc as plsc

N = 20
GROUP = 16


def _nsqrt(s):
    i = lax.bitcast_convert_type(s, jnp.int32)
    y = lax.bitcast_convert_type(jnp.int32(0x5F3759DF) - (i >> 1), jnp.float32)
    for _ in range(3):
        y = y * (1.5 - 0.5 * s * y * y)
    return s * y


def _sc_scalar_features(p3, B):
    info = plsc.get_sparse_core_info()
    NC, NS = info.num_cores, info.num_subcores
    NW = NC * NS
    n_groups = B // GROUP
    groups_per_w = n_groups // NW
    mesh = plsc.VectorSubcoreMesh(core_axis_name="c", subcore_axis_name="s")

    @functools.partial(
        pl.kernel,
        mesh=mesh,
        compiler_params=pltpu.CompilerParams(use_tc_tiling_on_sc=False),
        out_type=jax.ShapeDtypeStruct((N, 8, B), jnp.float32),
        scratch_types=[
            pltpu.VMEM((60, GROUP), jnp.float32),
            pltpu.VMEM((N, N, GROUP), jnp.float32),
            pltpu.VMEM((N, 8, GROUP), jnp.float32),
        ],
    )
    def sc_k(p_hbm, g_hbm, in_buf, d_buf, t_buf):
        wid = lax.axis_index("s") * NC + lax.axis_index("c")
        c_one = jnp.full((GROUP,), 1.0, jnp.float32)
        c_zero = jnp.zeros((GROUP,), jnp.float32)

        for i in range(N):
            t_buf[i, 5, :] = c_one
            t_buf[i, 6, :] = c_zero
            t_buf[i, 7, :] = c_zero

        def group_body(g, carry):
            gid = wid * groups_per_w + g
            b0 = gid * GROUP
            pltpu.sync_copy(p_hbm.at[gid], in_buf)

            cx = in_buf[0, :]
            cy = in_buf[N, :]
            cz = in_buf[2 * N, :]
            for i in range(1, N):
                cx = cx + in_buf[i, :]
                cy = cy + in_buf[N + i, :]
                cz = cz + in_buf[2 * N + i, :]
            inv_n = jnp.float32(1.0 / N)
            cx, cy, cz = cx * inv_n, cy * inv_n, cz * inv_n

            for i in range(N):
                for j in range(i + 1, N):
                    dx = in_buf[i, :] - in_buf[j, :]
                    dy = in_buf[N + i, :] - in_buf[N + j, :]
                    dz = in_buf[2 * N + i, :] - in_buf[2 * N + j, :]
                    s = dx * dx + dy * dy + dz * dz
                    d_buf[i, j, :] = s
                    d_buf[j, i, :] = s

            big = jnp.full((GROUP,), 1e30, jnp.float32)
            for i in range(N):
                m1, m2, m3 = big, big, big
                for j in range(N):
                    if j == i:
                        continue
                    v = d_buf[i, j, :]
                    t1 = jnp.minimum(m1, v)
                    v = jnp.maximum(m1, v)
                    m1 = t1
                    t2 = jnp.minimum(m2, v)
                    v = jnp.maximum(m2, v)
                    m2 = t2
                    m3 = jnp.minimum(m3, v)
                den = (_nsqrt(m1) + _nsqrt(m2) + _nsqrt(m3)) * jnp.float32(1.0 / 3.0)

                rx = in_buf[i, :] - cx
                ry = in_buf[N + i, :] - cy
                rz = in_buf[2 * N + i, :] - cz
                cd = _nsqrt(rx * rx + ry * ry + rz * rz)

                t_buf[i, 0, :] = rx
                t_buf[i, 1, :] = ry
                t_buf[i, 2, :] = rz
                t_buf[i, 3, :] = cd
                t_buf[i, 4, :] = den

            pltpu.sync_copy(t_buf, g_hbm.at[:, :, pl.ds(b0, GROUP)])
            return carry

        lax.fori_loop(0, groups_per_w, group_body, 0)

    return sc_k(p3)


def _tc_matmul(G, Wc, R):
    RT = 2048

    def body(g_ref, w_ref, o_ref):
        g = g_ref[...].astype(jnp.bfloat16)
        o_ref[...] = lax.dot_general(
            g, w_ref[...], (((1,), (0,)), ((), ())),
            preferred_element_type=jnp.float32)

    return pl.pallas_call(
        body,
        grid=(R // RT,),
        in_specs=[
            pl.BlockSpec((RT, 8), lambda i: (i, 0)),
            pl.BlockSpec((8, 256), lambda i: (0, 0)),
        ],
        out_specs=pl.BlockSpec((RT, 256), lambda i: (i, 0)),
        out_shape=jax.ShapeDtypeStruct((R, 256), jnp.float32),
    )(G, Wc)


def kernel(points, W_rel, b_rel, W_dist, b_dist, emb, W_den, b_den):
    B, n, _ = points.shape
    D4 = W_rel.shape[1]

    p2 = jnp.transpose(points, (2, 1, 0)).reshape(3 * n, B)
    p3 = p2.reshape(3 * n, B // GROUP, GROUP).transpose(1, 0, 2)

    Wc = jnp.zeros((8, 4 * D4), jnp.float32)
    Wc = Wc.at[0:3, 0:D4].set(W_rel)
    Wc = Wc.at[3, D4:2 * D4].set(W_dist[0])
    Wc = Wc.at[4, 3 * D4:4 * D4].set(W_den[0])
    Wc = Wc.at[5, 0:D4].set(b_rel)
    Wc = Wc.at[5, D4:2 * D4].set(b_dist)
    Wc = Wc.at[5, 2 * D4:3 * D4].set(emb[n])
    Wc = Wc.at[5, 3 * D4:4 * D4].set(b_den)

    g6 = _sc_scalar_features(p3, B)
    G = g6.transpose(2, 0, 1).reshape(B * n, 8)
    out = _tc_matmul(G, Wc.astype(jnp.bfloat16), B * n)
    return out.reshape(B, n, 4 * D4)


# speedup vs baseline: 15.4722x; 1.9548x over previous
import functools

import jax
import jax.numpy as jnp
from jax import lax
from jax.experimental import pallas as pl
from jax.experimental.pallas import tpu as pltpu
from jax.experimental.pallas import tpu_sc as plsc

N = 20
GROUP = 16


def _nsqrt(s):
    i = lax.bitcast_convert_type(s, jnp.int32)
    y = lax.bitcast_convert_type(jnp.int32(0x5F3759DF) - (i >> 1), jnp.float32)
    for _ in range(3):
        y = y * (1.5 - 0.5 * s * y * y)
    return s * y


def _sc_scalar_features(p3, B):
    info = plsc.get_sparse_core_info()
    NC, NS = info.num_cores, info.num_subcores
    NW = NC * NS
    SLAB = 8 * GROUP
    groups_per_w = (B // GROUP) // NW
    slabs_per_w = groups_per_w // 8
    mesh = plsc.VectorSubcoreMesh(core_axis_name="c", subcore_axis_name="s")

    @functools.partial(
        pl.kernel,
        mesh=mesh,
        out_type=jax.ShapeDtypeStruct((N, 8, B), jnp.float32),
        scratch_types=[
            pltpu.VMEM((60, SLAB), jnp.float32),
            pltpu.VMEM((N, N, GROUP), jnp.float32),
            pltpu.VMEM((N, 8, SLAB), jnp.float32),
        ],
    )
    def sc_k(p_hbm, g_hbm, in_buf, d_buf, t_buf):
        wid = lax.axis_index("s") * NC + lax.axis_index("c")
        c_one = jnp.full((GROUP,), 1.0, jnp.float32)
        c_zero = jnp.zeros((GROUP,), jnp.float32)

        for i in range(N):
            for c in range(8):
                t_buf[i, 5, pl.ds(c * GROUP, GROUP)] = c_one
                t_buf[i, 6, pl.ds(c * GROUP, GROUP)] = c_zero
                t_buf[i, 7, pl.ds(c * GROUP, GROUP)] = c_zero

        def group_body(g, carry):
            k = lax.rem(g, 8)
            slab = lax.div(g, 8)
            k16 = k * GROUP

            @pl.when(k == 0)
            def _load():
                pltpu.sync_copy(p_hbm.at[wid * slabs_per_w + slab], in_buf)

            def ld(r):
                return in_buf[r, pl.ds(k16, GROUP)]

            cx = ld(0)
            cy = ld(N)
            cz = ld(2 * N)
            for i in range(1, N):
                cx = cx + ld(i)
                cy = cy + ld(N + i)
                cz = cz + ld(2 * N + i)
            inv_n = jnp.float32(1.0 / N)
            cx, cy, cz = cx * inv_n, cy * inv_n, cz * inv_n

            for i in range(N):
                for j in range(i + 1, N):
                    dx = ld(i) - ld(j)
                    dy = ld(N + i) - ld(N + j)
                    dz = ld(2 * N + i) - ld(2 * N + j)
                    s = dx * dx + dy * dy + dz * dz
                    d_buf[i, j, :] = s
                    d_buf[j, i, :] = s

            big = jnp.full((GROUP,), 1e30, jnp.float32)
            for i in range(N):
                m1, m2, m3 = big, big, big
                for j in range(N):
                    if j == i:
                        continue
                    v = d_buf[i, j, :]
                    t1 = jnp.minimum(m1, v)
                    v = jnp.maximum(m1, v)
                    m1 = t1
                    t2 = jnp.minimum(m2, v)
                    v = jnp.maximum(m2, v)
                    m2 = t2
                    m3 = jnp.minimum(m3, v)
                den = (_nsqrt(m1) + _nsqrt(m2) + _nsqrt(m3)) * jnp.float32(1.0 / 3.0)

                rx = ld(i) - cx
                ry = ld(N + i) - cy
                rz = ld(2 * N + i) - cz
                cd = _nsqrt(rx * rx + ry * ry + rz * rz)

                t_buf[i, 0, pl.ds(k16, GROUP)] = rx
                t_buf[i, 1, pl.ds(k16, GROUP)] = ry
                t_buf[i, 2, pl.ds(k16, GROUP)] = rz
                t_buf[i, 3, pl.ds(k16, GROUP)] = cd
                t_buf[i, 4, pl.ds(k16, GROUP)] = den

            @pl.when(k == 7)
            def _flush():
                b0 = (wid * slabs_per_w + slab) * SLAB
                pltpu.sync_copy(t_buf, g_hbm.at[:, :, pl.ds(b0, SLAB)])

            return carry

        lax.fori_loop(0, groups_per_w, group_body, 0)

    return sc_k(p3)


def _tc_matmul(g6, Wc, B):
    BT = 2048

    def body(g_ref, w_ref, o_ref):
        g = g_ref[0].astype(jnp.bfloat16)
        o_ref[:, 0, 0, :] = lax.dot_general(
            g, w_ref[...], (((0,), (0,)), ((), ())),
            preferred_element_type=jnp.float32)

    out = pl.pallas_call(
        body,
        grid=(B // BT, N),
        in_specs=[
            pl.BlockSpec((1, 8, BT), lambda bc, i: (i, 0, bc)),
            pl.BlockSpec((8, 256), lambda bc, i: (0, 0)),
        ],
        out_specs=pl.BlockSpec((BT, 1, 1, 256), lambda bc, i: (bc, i, 0, 0)),
        out_shape=jax.ShapeDtypeStruct((B, N, 1, 256), jnp.float32),
    )(g6, Wc)
    return out.reshape(B, N, 256)


def kernel(points, W_rel, b_rel, W_dist, b_dist, emb, W_den, b_den):
    B, n, _ = points.shape
    D4 = W_rel.shape[1]

    SLAB = 8 * GROUP
    p2 = jnp.transpose(points, (2, 1, 0)).reshape(3 * n, B)
    p3 = p2.reshape(3 * n, B // SLAB, SLAB).transpose(1, 0, 2)

    Wc = jnp.zeros((8, 4 * D4), jnp.float32)
    Wc = Wc.at[0:3, 0:D4].set(W_rel)
    Wc = Wc.at[3, D4:2 * D4].set(W_dist[0])
    Wc = Wc.at[4, 3 * D4:4 * D4].set(W_den[0])
    Wc = Wc.at[5, 0:D4].set(b_rel)
    Wc = Wc.at[5, D4:2 * D4].set(b_dist)
    Wc = Wc.at[5, 2 * D4:3 * D4].set(emb[n])
    Wc = Wc.at[5, 3 * D4:4 * D4].set(b_den)

    g6 = _sc_scalar_features(p3, B)
    return _tc_matmul(g6, Wc.astype(jnp.bfloat16), B)


# speedup vs baseline: 19.8103x; 1.2804x over previous
import functools

import jax
import jax.numpy as jnp
from jax import lax
from jax.experimental import pallas as pl
from jax.experimental.pallas import tpu as pltpu
from jax.experimental.pallas import tpu_sc as plsc

N = 20
GROUP = 16


def _nsqrt(s):
    i = lax.bitcast_convert_type(s, jnp.int32)
    y = lax.bitcast_convert_type(jnp.int32(0x5F3759DF) - (i >> 1), jnp.float32)
    for _ in range(2):
        y = y * (1.5 - 0.5 * s * y * y)
    return s * y


def _sc_scalar_features(p3, B):
    info = plsc.get_sparse_core_info()
    NC, NS = info.num_cores, info.num_subcores
    NW = NC * NS
    SLAB = 8 * GROUP
    groups_per_w = (B // GROUP) // NW
    slabs_per_w = groups_per_w // 8
    mesh = plsc.VectorSubcoreMesh(core_axis_name="c", subcore_axis_name="s")

    @functools.partial(
        pl.kernel,
        mesh=mesh,
        out_type=jax.ShapeDtypeStruct((N, 8, B), jnp.float32),
        scratch_types=[
            pltpu.VMEM((60, SLAB), jnp.float32),
            pltpu.VMEM((N, N, GROUP), jnp.float32),
            pltpu.VMEM((N, 8, SLAB), jnp.float32),
        ],
    )
    def sc_k(p_hbm, g_hbm, in_buf, d_buf, t_buf):
        wid = lax.axis_index("s") * NC + lax.axis_index("c")
        c_one = jnp.full((GROUP,), 1.0, jnp.float32)
        c_zero = jnp.zeros((GROUP,), jnp.float32)

        for i in range(N):
            for c in range(8):
                t_buf[i, 5, pl.ds(c * GROUP, GROUP)] = c_one
                t_buf[i, 6, pl.ds(c * GROUP, GROUP)] = c_zero
                t_buf[i, 7, pl.ds(c * GROUP, GROUP)] = c_zero

        def group_body(g, carry):
            k = lax.rem(g, 8)
            slab = lax.div(g, 8)
            k16 = k * GROUP

            @pl.when(k == 0)
            def _load():
                pltpu.sync_copy(p_hbm.at[wid * slabs_per_w + slab], in_buf)

            def ld(r):
                return in_buf[r, pl.ds(k16, GROUP)]

            cx = ld(0)
            cy = ld(N)
            cz = ld(2 * N)
            for i in range(1, N):
                cx = cx + ld(i)
                cy = cy + ld(N + i)
                cz = cz + ld(2 * N + i)
            inv_n = jnp.float32(1.0 / N)
            cx, cy, cz = cx * inv_n, cy * inv_n, cz * inv_n

            IB = 4
            for i0 in range(0, N, IB):
                rows = []
                for i in range(i0, min(i0 + IB, N)):
                    rows.append((i, ld(i), ld(N + i), ld(2 * N + i)))
                for j in range(i0 + 1, N):
                    xj, yj, zj = ld(j), ld(N + j), ld(2 * N + j)
                    for (i, xi, yi, zi) in rows:
                        if i >= j:
                            continue
                        dx = xi - xj
                        dy = yi - yj
                        dz = zi - zj
                        d_buf[i, j, :] = dx * dx + dy * dy + dz * dz

            big = jnp.full((GROUP,), 1e30, jnp.float32)
            for i in range(N):
                m1, m2, m3 = big, big, big
                for j in range(N):
                    if j == i:
                        continue
                    v = d_buf[i, j, :] if i < j else d_buf[j, i, :]
                    t1 = jnp.minimum(m1, v)
                    v = jnp.maximum(m1, v)
                    m1 = t1
                    t2 = jnp.minimum(m2, v)
                    v = jnp.maximum(m2, v)
                    m2 = t2
                    m3 = jnp.minimum(m3, v)
                den = (_nsqrt(m1) + _nsqrt(m2) + _nsqrt(m3)) * jnp.float32(1.0 / 3.0)

                rx = ld(i) - cx
                ry = ld(N + i) - cy
                rz = ld(2 * N + i) - cz
                cd = _nsqrt(rx * rx + ry * ry + rz * rz)

                t_buf[i, 0, pl.ds(k16, GROUP)] = rx
                t_buf[i, 1, pl.ds(k16, GROUP)] = ry
                t_buf[i, 2, pl.ds(k16, GROUP)] = rz
                t_buf[i, 3, pl.ds(k16, GROUP)] = cd
                t_buf[i, 4, pl.ds(k16, GROUP)] = den

            @pl.when(k == 7)
            def _flush():
                b0 = (wid * slabs_per_w + slab) * SLAB
                pltpu.sync_copy(t_buf, g_hbm.at[:, :, pl.ds(b0, SLAB)])

            return carry

        lax.fori_loop(0, groups_per_w, group_body, 0)

    return sc_k(p3)


def _tc_transpose(pts2d, B):
    TS = 2048

    def body(x_ref, o_ref):
        o_ref[...] = x_ref[...].T

    return pl.pallas_call(
        body,
        grid=(B // TS,),
        in_specs=[pl.BlockSpec((TS, 60), lambda i: (i, 0))],
        out_specs=pl.BlockSpec((60, TS), lambda i: (0, i)),
        out_shape=jax.ShapeDtypeStruct((60, B), jnp.float32),
    )(pts2d)


def _tc_matmul(g6, Wc, B):
    BT = 4096

    def body(g_ref, w_ref, o_ref):
        g = g_ref[0].astype(jnp.bfloat16)
        o_ref[:, 0, 0, :] = lax.dot_general(
            g, w_ref[...], (((0,), (0,)), ((), ())),
            preferred_element_type=jnp.float32)

    out = pl.pallas_call(
        body,
        grid=(B // BT, N),
        in_specs=[
            pl.BlockSpec((1, 8, BT), lambda bc, i: (i, 0, bc)),
            pl.BlockSpec((8, 256), lambda bc, i: (0, 0)),
        ],
        out_specs=pl.BlockSpec((BT, 1, 1, 256), lambda bc, i: (bc, i, 0, 0)),
        out_shape=jax.ShapeDtypeStruct((B, N, 1, 256), jnp.float32),
    )(g6, Wc)
    return out.reshape(B, N, 256)


def kernel(points, W_rel, b_rel, W_dist, b_dist, emb, W_den, b_den):
    B, n, _ = points.shape
    D4 = W_rel.shape[1]

    SLAB = 8 * GROUP
    p2 = jnp.transpose(points, (2, 1, 0)).reshape(3 * n, B)
    p3 = p2.reshape(3 * n, B // SLAB, SLAB).transpose(1, 0, 2)

    Wc = jnp.zeros((8, 4 * D4), jnp.float32)
    Wc = Wc.at[0:3, 0:D4].set(W_rel)
    Wc = Wc.at[3, D4:2 * D4].set(W_dist[0])
    Wc = Wc.at[4, 3 * D4:4 * D4].set(W_den[0])
    Wc = Wc.at[5, 0:D4].set(b_rel)
    Wc = Wc.at[5, D4:2 * D4].set(b_dist)
    Wc = Wc.at[5, 2 * D4:3 * D4].set(emb[n])
    Wc = Wc.at[5, 3 * D4:4 * D4].set(b_den)

    g6 = _sc_scalar_features(p3, B)
    return _tc_matmul(g6, Wc.astype(jnp.bfloat16), B)
